# baseline (speedup 1.0000x reference)
import functools

import jax
import jax.numpy as jnp
import numpy as np
from jax import lax
from jax.experimental import pallas as pl
from jax.experimental.pallas import tpu as pltpu
from jax.experimental.pallas import tpu_sc as plsc

T, B, D = 8192, 4, 1024
N = T * B
NC, NS, L = 2, 16, 16
NW = NC * NS
T_SC = T // 2
N_SC = T_SC * B
T_PER_W = T_SC // NW
KT = 4
K = KT * B
NCHUNK = T_PER_W // KT
NBUF = 2
TBLK = 256
N_BASE = 10000.0


def _cumsum_body(f_ref, idx_ref):
    f = f_ref[...]
    t = lax.broadcasted_iota(jnp.int32, (T, B), 0)
    c = jnp.where(t == 0, 0, f.astype(jnp.int32))
    k = 1
    while k < T:
        z = jnp.zeros((k, B), jnp.int32)
        c = c + jnp.concatenate([z, c[:-k, :]], axis=0)
        k *= 2
    idx_ref[...] = c


def _cumsum(flags):
    return pl.pallas_call(
        _cumsum_body,
        out_shape=jax.ShapeDtypeStruct((T, B), jnp.int32),
    )(flags)


def _sc_body(idx_hbm, x_hbm, tbl_hbm, out_hbm, idx_v, *bufs):
    pe = bufs[0:NBUF]
    xb = bufs[NBUF:2 * NBUF]
    gsem = bufs[2 * NBUF:3 * NBUF]
    xsem = bufs[3 * NBUF:4 * NBUF]
    osem = bufs[4 * NBUF:5 * NBUF]
    cc = lax.axis_index("c")
    ss = lax.axis_index("s")
    wid = ss * NC + cc
    t0 = wid * T_PER_W
    base = t0 * B

    pltpu.sync_copy(idx_hbm.at[pl.ds(base, T_PER_W * B)], idx_v)

    def issue_in(g, b):
        pltpu.async_copy(tbl_hbm.at[idx_v.at[pl.ds(g * K, K)]], pe[b], gsem[b])
        pltpu.async_copy(x_hbm.at[pl.ds(t0 + g * KT, KT)], xb[b], xsem[b])

    def process(g, b, prefetch_b):
        @pl.when(g >= 1)
        def _():
            pltpu.make_async_copy(xb[prefetch_b], out_hbm.at[pl.ds(t0, KT)],
                                  osem[prefetch_b]).wait()

        @pl.when(g + NBUF - 1 < NCHUNK)
        def _():
            issue_in(g + NBUF - 1, prefetch_b)

        pltpu.make_async_copy(x_hbm.at[pl.ds(t0, KT)], pe[b], gsem[b]).wait()
        pltpu.make_async_copy(x_hbm.at[pl.ds(t0, KT)], xb[b], xsem[b]).wait()

        def add_row(r, _):
            t = r // B
            bb = lax.rem(r, B)
            for j in range(D // L):
                sl = pl.ds(j * L, L)
                plsc.addupdate(xb[b].at[t, bb, sl], pe[b][r, sl])
            return 0

        lax.fori_loop(0, K, add_row, 0)
        pltpu.async_copy(xb[b], out_hbm.at[pl.ds(t0 + g * KT, KT)], osem[b])

    for b in range(NBUF - 1):
        issue_in(b, b)

    def group(p, _):
        for b in range(NBUF):
            g = p * NBUF + b
            process(g, b, (b + NBUF - 1) % NBUF)
        return 0

    lax.fori_loop(0, NCHUNK // NBUF, group, 0)
    for g in range(NCHUNK - NCHUNK % NBUF, NCHUNK):
        process(g, g % NBUF, (g + NBUF - 1) % NBUF)
    bl = (NCHUNK - 1) % NBUF
    pltpu.make_async_copy(xb[bl], out_hbm.at[pl.ds(t0, KT)], osem[bl]).wait()


def _sc_gather_add(idx, x, tbl):
    mesh = plsc.VectorSubcoreMesh(core_axis_name="c", subcore_axis_name="s")
    fn = functools.partial(
        pl.kernel,
        mesh=mesh,
        out_type=jax.ShapeDtypeStruct((T, B, D), jnp.float32),
        scratch_types=(
            [pltpu.VMEM((T_PER_W * B,), jnp.int32)]
            + [pltpu.VMEM((K, D), jnp.float32) for _ in range(NBUF)]
            + [pltpu.VMEM((KT, B, D), jnp.float32) for _ in range(NBUF)]
            + [pltpu.SemaphoreType.DMA for _ in range(3 * NBUF)]
        ),
    )(_sc_body)
    return fn(idx, x, tbl)


def _sincos_body(c_ref, x_ref, _, out_ref):
    d = lax.broadcasted_iota(jnp.int32, (1, 1, D), 2)
    k2 = (d >> 1 << 1).astype(jnp.float32)
    div = jnp.exp(k2 * (-np.log(N_BASE) / D))
    phase = (d & 1).astype(jnp.float32) * (np.pi / 2.0)
    c = c_ref[...].astype(jnp.float32)[:, :, None]
    out_ref[...] = x_ref[...] + jnp.sin(c * div + phase)


def _sincos_add(cum, x, sc_out):
    nblk = (T - T_SC) // TBLK
    return pl.pallas_call(
        _sincos_body,
        grid=(nblk,),
        in_specs=[
            pl.BlockSpec((TBLK, B), lambda i: (T_SC // TBLK + i, 0)),
            pl.BlockSpec((TBLK, B, D), lambda i: (T_SC // TBLK + i, 0, 0)),
            pl.BlockSpec(memory_space=pl.ANY),
        ],
        out_specs=pl.BlockSpec((TBLK, B, D), lambda i: (T_SC // TBLK + i, 0, 0)),
        out_shape=jax.ShapeDtypeStruct((T, B, D), jnp.float32),
        input_output_aliases={2: 0},
    )(cum, x, sc_out)


def kernel(x_original, x_projected_to_d_model, pe_table):
    flags = x_original[:, :, -1]
    cum = _cumsum(flags)
    idx = cum[:T_SC].reshape(N_SC)
    sc_out = _sc_gather_add(idx, x_projected_to_d_model, pe_table)
    return _sincos_add(cum, x_projected_to_d_model, sc_out)


# speedup vs baseline: 2.2937x; 2.2937x over previous
import functools

import jax
import jax.numpy as jnp
import numpy as np
from jax import lax
from jax.experimental import pallas as pl
from jax.experimental.pallas import tpu as pltpu
from jax.experimental.pallas import tpu_sc as plsc

T, B, D = 8192, 4, 1024
N = T * B
NC, NS, L = 2, 16, 16
NW = NC * NS
T_SC = 6400
N_SC = T_SC * B
T_PER_W = T_SC // NW
KT = 4
K = KT * B
NCHUNK = T_PER_W // KT
NBUF = 2
TBLK = 256
N_BASE = 10000.0


def _cumsum_body(f_ref, idx_ref):
    f = f_ref[...]
    t = lax.broadcasted_iota(jnp.int32, (T, B), 0)
    c = jnp.where(t == 0, 0, f.astype(jnp.int32))
    k = 1
    while k < T:
        z = jnp.zeros((k, B), jnp.int32)
        c = c + jnp.concatenate([z, c[:-k, :]], axis=0)
        k *= 2
    idx_ref[...] = c


def _cumsum(flags):
    return pl.pallas_call(
        _cumsum_body,
        out_shape=jax.ShapeDtypeStruct((T, B), jnp.int32),
    )(flags)


def _sc_body(idx_hbm, x_hbm, tbl_hbm, out_hbm, idx_v, *bufs):
    pe = bufs[0:NBUF]
    xb = bufs[NBUF:2 * NBUF]
    gsem = bufs[2 * NBUF:3 * NBUF]
    xsem = bufs[3 * NBUF:4 * NBUF]
    osem = bufs[4 * NBUF:5 * NBUF]
    cc = lax.axis_index("c")
    ss = lax.axis_index("s")
    wid = ss * NC + cc
    t0 = wid * T_PER_W
    base = t0 * B

    pltpu.sync_copy(idx_hbm.at[pl.ds(base, T_PER_W * B)], idx_v)

    def issue_in(g, b):
        pltpu.async_copy(tbl_hbm.at[idx_v.at[pl.ds(g * K, K)]], pe[b], gsem[b])
        pltpu.async_copy(x_hbm.at[pl.ds(t0 + g * KT, KT)], xb[b], xsem[b])

    def process(g, b, prefetch_b):
        @pl.when(g >= 1)
        def _():
            pltpu.make_async_copy(xb[prefetch_b], out_hbm.at[pl.ds(t0, KT)],
                                  osem[prefetch_b]).wait()

        @pl.when(g + NBUF - 1 < NCHUNK)
        def _():
            issue_in(g + NBUF - 1, prefetch_b)

        pltpu.make_async_copy(x_hbm.at[pl.ds(t0, KT)], pe[b], gsem[b]).wait()
        pltpu.make_async_copy(x_hbm.at[pl.ds(t0, KT)], xb[b], xsem[b]).wait()

        def add_row(r, _):
            t = r // B
            bb = lax.rem(r, B)
            for j in range(D // L):
                sl = pl.ds(j * L, L)
                plsc.addupdate(xb[b].at[t, bb, sl], pe[b][r, sl])
            return 0

        lax.fori_loop(0, K, add_row, 0)
        pltpu.async_copy(xb[b], out_hbm.at[pl.ds(t0 + g * KT, KT)], osem[b])

    for b in range(NBUF - 1):
        issue_in(b, b)

    def group(p, _):
        for b in range(NBUF):
            g = p * NBUF + b
            process(g, b, (b + NBUF - 1) % NBUF)
        return 0

    lax.fori_loop(0, NCHUNK // NBUF, group, 0)
    for g in range(NCHUNK - NCHUNK % NBUF, NCHUNK):
        process(g, g % NBUF, (g + NBUF - 1) % NBUF)
    bl = (NCHUNK - 1) % NBUF
    pltpu.make_async_copy(xb[bl], out_hbm.at[pl.ds(t0, KT)], osem[bl]).wait()


def _sc_gather_add(idx, x, tbl):
    mesh = plsc.VectorSubcoreMesh(core_axis_name="c", subcore_axis_name="s")
    fn = functools.partial(
        pl.kernel,
        mesh=mesh,
        out_type=jax.ShapeDtypeStruct((T, B, D), jnp.float32),
        scratch_types=(
            [pltpu.VMEM((T_PER_W * B,), jnp.int32)]
            + [pltpu.VMEM((K, D), jnp.float32) for _ in range(NBUF)]
            + [pltpu.VMEM((KT, B, D), jnp.float32) for _ in range(NBUF)]
            + [pltpu.SemaphoreType.DMA for _ in range(3 * NBUF)]
        ),
    )(_sc_body)
    return fn(idx, x, tbl)


def _sincos_body(c_ref, x_ref, out_ref):
    d = lax.broadcasted_iota(jnp.int32, (1, 1, D), 2)
    k2 = (d >> 1 << 1).astype(jnp.float32)
    div = jnp.exp(k2 * (-np.log(N_BASE) / D))
    phase = (d & 1).astype(jnp.float32) * (np.pi / 2.0)
    c = c_ref[...].astype(jnp.float32)[:, :, None]
    out_ref[...] = x_ref[...] + jnp.sin(c * div + phase)


def _sincos_add(cum, x):
    nblk = (T - T_SC) // TBLK
    return pl.pallas_call(
        _sincos_body,
        grid=(nblk,),
        in_specs=[
            pl.BlockSpec((TBLK, B), lambda i: (T_SC // TBLK + i, 0)),
            pl.BlockSpec((TBLK, B, D), lambda i: (T_SC // TBLK + i, 0, 0)),
        ],
        out_specs=pl.BlockSpec((TBLK, B, D), lambda i: (i, 0, 0)),
        out_shape=jax.ShapeDtypeStruct((T - T_SC, B, D), jnp.float32),
    )(cum, x)


def kernel(x_original, x_projected_to_d_model, pe_table):
    flags = x_original[:, :, -1]
    cum = _cumsum(flags)
    idx = cum[:T_SC].reshape(N_SC)
    sc_out = _sc_gather_add(idx, x_projected_to_d_model, pe_table)
    tc_half = _sincos_add(cum, x_projected_to_d_model)
    return lax.dynamic_update_slice(sc_out, tc_half, (T_SC, 0, 0))


# speedup vs baseline: 2.3482x; 1.0238x over previous
import functools

import jax
import jax.numpy as jnp
import numpy as np
from jax import lax
from jax.experimental import pallas as pl
from jax.experimental.pallas import tpu as pltpu
from jax.experimental.pallas import tpu_sc as plsc

T, B, D = 8192, 4, 1024
N = T * B
NC, NS, L = 2, 16, 16
NW = NC * NS
T_SC = 6400
N_SC = T_SC * B
T_PER_W = T_SC // NW
KT = 4
K = KT * B
NCHUNK = T_PER_W // KT
NBUF = 2
TBLK = 256
N_BASE = 10000.0


def _cumsum_body(f_ref, idx_ref):
    f = f_ref[...]
    t = lax.broadcasted_iota(jnp.int32, (T, B), 0)
    c = jnp.where(t == 0, 0, f.astype(jnp.int32))
    k = 1
    while k < T:
        z = jnp.zeros((k, B), jnp.int32)
        c = c + jnp.concatenate([z, c[:-k, :]], axis=0)
        k *= 2
    idx_ref[...] = c


def _cumsum(flags):
    return pl.pallas_call(
        _cumsum_body,
        out_shape=jax.ShapeDtypeStruct((T, B), jnp.int32),
    )(flags)


def _sc_body(idx_hbm, x_hbm, tbl_hbm, out_hbm, idx_v, *bufs):
    pe = bufs[0:NBUF]
    xb = bufs[NBUF:2 * NBUF]
    gsem = bufs[2 * NBUF:3 * NBUF]
    xsem = bufs[3 * NBUF:4 * NBUF]
    osem = bufs[4 * NBUF:5 * NBUF]
    cc = lax.axis_index("c")
    ss = lax.axis_index("s")
    wid = ss * NC + cc
    t0 = wid * T_PER_W
    base = t0 * B

    pltpu.sync_copy(idx_hbm.at[pl.ds(base, T_PER_W * B)], idx_v)

    def issue_in(g, b):
        pltpu.async_copy(tbl_hbm.at[idx_v.at[pl.ds(g * K, K)]], pe[b], gsem[b])
        pltpu.async_copy(x_hbm.at[pl.ds(t0 + g * KT, KT)], xb[b], xsem[b])

    def process(g, b, prefetch_b):
        @pl.when(g >= 1)
        def _():
            pltpu.make_async_copy(xb[prefetch_b], out_hbm.at[pl.ds(t0, KT)],
                                  osem[prefetch_b]).wait()

        @pl.when(g + NBUF - 1 < NCHUNK)
        def _():
            issue_in(g + NBUF - 1, prefetch_b)

        pltpu.make_async_copy(x_hbm.at[pl.ds(t0, KT)], pe[b], gsem[b]).wait()
        pltpu.make_async_copy(x_hbm.at[pl.ds(t0, KT)], xb[b], xsem[b]).wait()

        def add_row(r, _):
            t = r // B
            bb = lax.rem(r, B)
            for j in range(D // L):
                sl = pl.ds(j * L, L)
                plsc.addupdate(xb[b].at[t, bb, sl], pe[b][r, sl])
            return 0

        lax.fori_loop(0, K, add_row, 0)
        pltpu.async_copy(xb[b], out_hbm.at[pl.ds(t0 + g * KT, KT)], osem[b])

    for b in range(NBUF - 1):
        issue_in(b, b)

    def group(p, _):
        for b in range(NBUF):
            g = p * NBUF + b
            process(g, b, (b + NBUF - 1) % NBUF)
        return 0

    lax.fori_loop(0, NCHUNK // NBUF, group, 0)
    for g in range(NCHUNK - NCHUNK % NBUF, NCHUNK):
        process(g, g % NBUF, (g + NBUF - 1) % NBUF)
    bl = (NCHUNK - 1) % NBUF
    pltpu.make_async_copy(xb[bl], out_hbm.at[pl.ds(t0, KT)], osem[bl]).wait()


def _sc_gather_add(idx, x, tbl):
    mesh = plsc.VectorSubcoreMesh(core_axis_name="c", subcore_axis_name="s")
    fn = functools.partial(
        pl.kernel,
        mesh=mesh,
        out_type=jax.ShapeDtypeStruct((T, B, D), jnp.float32),
        scratch_types=(
            [pltpu.VMEM((T_PER_W * B,), jnp.int32)]
            + [pltpu.VMEM((K, D), jnp.float32) for _ in range(NBUF)]
            + [pltpu.VMEM((KT, B, D), jnp.float32) for _ in range(NBUF)]
            + [pltpu.SemaphoreType.DMA for _ in range(3 * NBUF)]
        ),
    )(_sc_body)
    return fn(idx, x, tbl)


def _sincos_body(c_ref, x_ref, out_ref):
    d = lax.broadcasted_iota(jnp.int32, (1, D), 1)
    k2 = (d >> 1 << 1).astype(jnp.float32)
    div = jnp.exp(k2 * (-np.log(N_BASE) / D))
    phase = (d & 1).astype(jnp.float32) * (np.pi / 2.0)
    for bb in range(B):
        c = c_ref[:, bb].astype(jnp.float32)[:, None]
        out_ref[:, bb, :] = x_ref[:, bb, :] + jnp.sin(c * div + phase)


def _sincos_add(cum, x):
    nblk = (T - T_SC) // TBLK
    return pl.pallas_call(
        _sincos_body,
        grid=(nblk,),
        in_specs=[
            pl.BlockSpec((TBLK, B), lambda i: (T_SC // TBLK + i, 0)),
            pl.BlockSpec((TBLK, B, D), lambda i: (T_SC // TBLK + i, 0, 0)),
        ],
        out_specs=pl.BlockSpec((TBLK, B, D), lambda i: (i, 0, 0)),
        out_shape=jax.ShapeDtypeStruct((T - T_SC, B, D), jnp.float32),
    )(cum, x)


def kernel(x_original, x_projected_to_d_model, pe_table):
    flags = x_original[:, :, -1]
    cum = _cumsum(flags)
    idx = cum[:T_SC].reshape(N_SC)
    sc_out = _sc_gather_add(idx, x_projected_to_d_model, pe_table)
    tc_half = _sincos_add(cum, x_projected_to_d_model)
    return lax.dynamic_update_slice(sc_out, tc_half, (T_SC, 0, 0))


# speedup vs baseline: 2.5199x; 1.0731x over previous
import functools

import jax
import jax.numpy as jnp
import numpy as np
from jax import lax
from jax.experimental import pallas as pl
from jax.experimental.pallas import tpu as pltpu
from jax.experimental.pallas import tpu_sc as plsc

T, B, D = 8192, 4, 1024
N = T * B
NC, NS, L = 2, 16, 16
NW = NC * NS
T_SC = 5120
N_SC = T_SC * B
T_PER_W = T_SC // NW
KT = 4
K = KT * B
NCHUNK = T_PER_W // KT
NBUF = 2
TBLK = 256
N_BASE = 10000.0


def _cumsum_body(f_ref, idx_ref):
    f = f_ref[...]
    t = lax.broadcasted_iota(jnp.int32, (T, B), 0)
    c = jnp.where(t == 0, 0, f.astype(jnp.int32))
    k = 1
    while k < T:
        z = jnp.zeros((k, B), jnp.int32)
        c = c + jnp.concatenate([z, c[:-k, :]], axis=0)
        k *= 2
    idx_ref[...] = c


def _cumsum(flags):
    return pl.pallas_call(
        _cumsum_body,
        out_shape=jax.ShapeDtypeStruct((T, B), jnp.int32),
    )(flags)


def _sc_body(idx_hbm, x_hbm, tbl_hbm, out_hbm, idx_v, *bufs):
    pe = bufs[0:NBUF]
    xb = bufs[NBUF:2 * NBUF]
    gsem = bufs[2 * NBUF:3 * NBUF]
    xsem = bufs[3 * NBUF:4 * NBUF]
    osem = bufs[4 * NBUF:5 * NBUF]
    cc = lax.axis_index("c")
    ss = lax.axis_index("s")
    wid = ss * NC + cc
    t0 = wid * T_PER_W
    base = t0 * B

    pltpu.sync_copy(idx_hbm.at[pl.ds(base, T_PER_W * B)], idx_v)

    def issue_in(g, b):
        pltpu.async_copy(tbl_hbm.at[idx_v.at[pl.ds(g * K, K)]], pe[b], gsem[b])
        pltpu.async_copy(x_hbm.at[pl.ds(t0 + g * KT, KT)], xb[b], xsem[b])

    def process(g, b, prefetch_b):
        @pl.when(g >= 1)
        def _():
            pltpu.make_async_copy(xb[prefetch_b], out_hbm.at[pl.ds(t0, KT)],
                                  osem[prefetch_b]).wait()

        @pl.when(g + NBUF - 1 < NCHUNK)
        def _():
            issue_in(g + NBUF - 1, prefetch_b)

        pltpu.make_async_copy(x_hbm.at[pl.ds(t0, KT)], pe[b], gsem[b]).wait()
        pltpu.make_async_copy(x_hbm.at[pl.ds(t0, KT)], xb[b], xsem[b]).wait()

        def add_row(r, _):
            t = r // B
            bb = lax.rem(r, B)
            for j in range(D // L):
                sl = pl.ds(j * L, L)
                plsc.addupdate(xb[b].at[t, bb, sl], pe[b][r, sl])
            return 0

        lax.fori_loop(0, K, add_row, 0)
        pltpu.async_copy(xb[b], out_hbm.at[pl.ds(t0 + g * KT, KT)], osem[b])

    for b in range(NBUF - 1):
        issue_in(b, b)

    def group(p, _):
        for b in range(NBUF):
            g = p * NBUF + b
            process(g, b, (b + NBUF - 1) % NBUF)
        return 0

    lax.fori_loop(0, NCHUNK // NBUF, group, 0)
    for g in range(NCHUNK - NCHUNK % NBUF, NCHUNK):
        process(g, g % NBUF, (g + NBUF - 1) % NBUF)
    bl = (NCHUNK - 1) % NBUF
    pltpu.make_async_copy(xb[bl], out_hbm.at[pl.ds(t0, KT)], osem[bl]).wait()


def _sc_gather_add(idx, x, tbl):
    mesh = plsc.VectorSubcoreMesh(core_axis_name="c", subcore_axis_name="s")
    fn = functools.partial(
        pl.kernel,
        mesh=mesh,
        out_type=jax.ShapeDtypeStruct((T, B, D), jnp.float32),
        scratch_types=(
            [pltpu.VMEM((T_PER_W * B,), jnp.int32)]
            + [pltpu.VMEM((K, D), jnp.float32) for _ in range(NBUF)]
            + [pltpu.VMEM((KT, B, D), jnp.float32) for _ in range(NBUF)]
            + [pltpu.SemaphoreType.DMA for _ in range(3 * NBUF)]
        ),
    )(_sc_body)
    return fn(idx, x, tbl)


def _sincos_body(c_ref, x_ref, out_ref):
    d = lax.broadcasted_iota(jnp.int32, (1, D), 1)
    k2 = (d >> 1 << 1).astype(jnp.float32)
    div = jnp.exp(k2 * (-np.log(N_BASE) / D))
    phase = (d & 1).astype(jnp.float32) * (np.pi / 2.0)
    for bb in range(B):
        c = c_ref[:, bb].astype(jnp.float32)[:, None]
        out_ref[:, bb, :] = x_ref[:, bb, :] + jnp.sin(c * div + phase)


def _sincos_add(cum, x):
    nblk = (T - T_SC) // TBLK
    return pl.pallas_call(
        _sincos_body,
        grid=(nblk,),
        in_specs=[
            pl.BlockSpec((TBLK, B), lambda i: (T_SC // TBLK + i, 0)),
            pl.BlockSpec((TBLK, B, D), lambda i: (T_SC // TBLK + i, 0, 0)),
        ],
        out_specs=pl.BlockSpec((TBLK, B, D), lambda i: (i, 0, 0)),
        out_shape=jax.ShapeDtypeStruct((T - T_SC, B, D), jnp.float32),
    )(cum, x)


def kernel(x_original, x_projected_to_d_model, pe_table):
    flags = x_original[:, :, -1]
    cum = _cumsum(flags)
    idx = cum[:T_SC].reshape(N_SC)
    sc_out = _sc_gather_add(idx, x_projected_to_d_model, pe_table)
    tc_half = _sincos_add(cum, x_projected_to_d_model)
    return lax.dynamic_update_slice(sc_out, tc_half, (T_SC, 0, 0))


# speedup vs baseline: 2.5608x; 1.0162x over previous
import functools

import jax
import jax.numpy as jnp
import numpy as np
from jax import lax
from jax.experimental import pallas as pl
from jax.experimental.pallas import tpu as pltpu
from jax.experimental.pallas import tpu_sc as plsc

T, B, D = 8192, 4, 1024
N = T * B
NC, NS, L = 2, 16, 16
NW = NC * NS
T_TC = 3072
T_SC = T - T_TC
N_SC = T_SC * B
T_PER_W = T_SC // NW
KT = 4
K = KT * B
NCHUNK = T_PER_W // KT
NBUF = 2
TBLK = 256
N_BASE = 10000.0


def _cumsum_body(f_ref, idx_ref):
    f = f_ref[...]
    t = lax.broadcasted_iota(jnp.int32, (T, B), 0)
    c = jnp.where(t == 0, 0, f.astype(jnp.int32))
    k = 1
    while k < T:
        z = jnp.zeros((k, B), jnp.int32)
        c = c + jnp.concatenate([z, c[:-k, :]], axis=0)
        k *= 2
    idx_ref[...] = c


def _cumsum(flags):
    return pl.pallas_call(
        _cumsum_body,
        out_shape=jax.ShapeDtypeStruct((T, B), jnp.int32),
    )(flags)


def _sc_body(idx_hbm, x_hbm, tbl_hbm, out_hbm, idx_v, *bufs):
    pe = bufs[0:NBUF]
    xb = bufs[NBUF:2 * NBUF]
    gsem = bufs[2 * NBUF:3 * NBUF]
    xsem = bufs[3 * NBUF:4 * NBUF]
    osem = bufs[4 * NBUF:5 * NBUF]
    cc = lax.axis_index("c")
    ss = lax.axis_index("s")
    wid = ss * NC + cc
    t0 = T_TC + wid * T_PER_W
    base = (t0 - T_TC) * B

    pltpu.sync_copy(idx_hbm.at[pl.ds(base, T_PER_W * B)], idx_v)

    def issue_in(g, b):
        pltpu.async_copy(tbl_hbm.at[idx_v.at[pl.ds(g * K, K)]], pe[b], gsem[b])
        pltpu.async_copy(x_hbm.at[pl.ds(t0 + g * KT, KT)], xb[b], xsem[b])

    def process(g, b, prefetch_b):
        @pl.when(g >= 1)
        def _():
            pltpu.make_async_copy(xb[prefetch_b], out_hbm.at[pl.ds(t0, KT)],
                                  osem[prefetch_b]).wait()

        @pl.when(g + NBUF - 1 < NCHUNK)
        def _():
            issue_in(g + NBUF - 1, prefetch_b)

        pltpu.make_async_copy(x_hbm.at[pl.ds(t0, KT)], pe[b], gsem[b]).wait()
        pltpu.make_async_copy(x_hbm.at[pl.ds(t0, KT)], xb[b], xsem[b]).wait()

        def add_row(r, _):
            t = r // B
            bb = lax.rem(r, B)
            for j in range(D // L):
                sl = pl.ds(j * L, L)
                plsc.addupdate(xb[b].at[t, bb, sl], pe[b][r, sl])
            return 0

        lax.fori_loop(0, K, add_row, 0)
        pltpu.async_copy(xb[b], out_hbm.at[pl.ds(t0 + g * KT, KT)], osem[b])

    for b in range(NBUF - 1):
        issue_in(b, b)

    def group(p, _):
        for b in range(NBUF):
            g = p * NBUF + b
            process(g, b, (b + NBUF - 1) % NBUF)
        return 0

    lax.fori_loop(0, NCHUNK // NBUF, group, 0)
    for g in range(NCHUNK - NCHUNK % NBUF, NCHUNK):
        process(g, g % NBUF, (g + NBUF - 1) % NBUF)
    bl = (NCHUNK - 1) % NBUF
    pltpu.make_async_copy(xb[bl], out_hbm.at[pl.ds(t0, KT)], osem[bl]).wait()


def _sc_gather_add(idx, x, tbl):
    mesh = plsc.VectorSubcoreMesh(core_axis_name="c", subcore_axis_name="s")
    fn = functools.partial(
        pl.kernel,
        mesh=mesh,
        out_type=jax.ShapeDtypeStruct((T, B, D), jnp.float32),
        scratch_types=(
            [pltpu.VMEM((T_PER_W * B,), jnp.int32)]
            + [pltpu.VMEM((K, D), jnp.float32) for _ in range(NBUF)]
            + [pltpu.VMEM((KT, B, D), jnp.float32) for _ in range(NBUF)]
            + [pltpu.SemaphoreType.DMA for _ in range(3 * NBUF)]
        ),
    )(_sc_body)
    return fn(idx, x, tbl)


TSPAN = TBLK + 8


def _aligned_base(s):
    return pl.multiple_of((s >> 3) << 3, 8)


def _mm_body(starts_sm, c_ref, x_ref, tbl_ref, out_ref, slc, sems):
    i = pl.program_id(0)
    nblk = pl.num_programs(0)

    def issue(j, buf):
        for bb in range(B):
            c0 = _aligned_base(starts_sm[j, bb])
            pltpu.make_async_copy(tbl_ref.at[pl.ds(c0, TSPAN)],
                                  slc.at[buf, bb], sems.at[buf, bb]).start()

    @pl.when(i == 0)
    def _():
        issue(0, 0)

    @pl.when(i + 1 < nblk)
    def _():
        issue(i + 1, (i + 1) % 2)

    buf = i % 2
    for bb in range(B):
        pltpu.make_async_copy(tbl_ref.at[pl.ds(0, TSPAN)],
                              slc.at[buf, bb], sems.at[buf, bb]).wait()
    sidx = lax.broadcasted_iota(jnp.int32, (TBLK, TSPAN), 1)
    for bb in range(B):
        c0 = _aligned_base(starts_sm[i, bb])
        local = c_ref[:, bb] - c0
        onehot = (local[:, None] == sidx).astype(jnp.float32)
        pe = jnp.dot(onehot, slc[buf, bb],
                     preferred_element_type=jnp.float32)
        out_ref[:, bb, :] = x_ref[:, bb, :] + pe


def _mm_gather_add(starts, cum, x, tbl):
    nblk = T_TC // TBLK
    grid_spec = pltpu.PrefetchScalarGridSpec(
        num_scalar_prefetch=1,
        grid=(nblk,),
        in_specs=[
            pl.BlockSpec((TBLK, B), lambda i, s: (i, 0)),
            pl.BlockSpec((TBLK, B, D), lambda i, s: (i, 0, 0)),
            pl.BlockSpec(memory_space=pl.ANY),
        ],
        out_specs=pl.BlockSpec((TBLK, B, D), lambda i, s: (i, 0, 0)),
        scratch_shapes=[
            pltpu.VMEM((2, B, TSPAN, D), jnp.float32),
            pltpu.SemaphoreType.DMA((2, B)),
        ],
    )
    return pl.pallas_call(
        _mm_body,
        grid_spec=grid_spec,
        out_shape=jax.ShapeDtypeStruct((T_TC, B, D), jnp.float32),
    )(starts, cum, x, tbl)


def kernel(x_original, x_projected_to_d_model, pe_table):
    flags = x_original[:, :, -1]
    cum = _cumsum(flags)
    idx = cum[T_TC:].reshape(N_SC)
    sc_out = _sc_gather_add(idx, x_projected_to_d_model, pe_table)
    starts = cum[:T_TC:TBLK, :]
    tc_half = _mm_gather_add(starts, cum, x_projected_to_d_model, pe_table)
    return lax.dynamic_update_slice(sc_out, tc_half, (0, 0, 0))


# speedup vs baseline: 2.8288x; 1.1046x over previous
import functools

import jax
import jax.numpy as jnp
import numpy as np
from jax import lax
from jax.experimental import pallas as pl
from jax.experimental.pallas import tpu as pltpu
from jax.experimental.pallas import tpu_sc as plsc

T, B, D = 8192, 4, 1024
N = T * B
NC, NS, L = 2, 16, 16
NW = NC * NS
T_TC = 4096
T_SC = T - T_TC
N_SC = T_SC * B
T_PER_W = T_SC // NW
KT = 4
K = KT * B
NCHUNK = T_PER_W // KT
NBUF = 2
TBLK = 256
N_BASE = 10000.0


def _cumsum_body(f_ref, idx_ref):
    f = f_ref[...]
    t = lax.broadcasted_iota(jnp.int32, (T, B), 0)
    c = jnp.where(t == 0, 0, f.astype(jnp.int32))
    k = 1
    while k < T:
        z = jnp.zeros((k, B), jnp.int32)
        c = c + jnp.concatenate([z, c[:-k, :]], axis=0)
        k *= 2
    idx_ref[...] = c


def _cumsum(flags):
    return pl.pallas_call(
        _cumsum_body,
        out_shape=jax.ShapeDtypeStruct((T, B), jnp.int32),
    )(flags)


def _sc_body(idx_hbm, x_hbm, tbl_hbm, out_hbm, idx_v, *bufs):
    pe = bufs[0:NBUF]
    xb = bufs[NBUF:2 * NBUF]
    gsem = bufs[2 * NBUF:3 * NBUF]
    xsem = bufs[3 * NBUF:4 * NBUF]
    osem = bufs[4 * NBUF:5 * NBUF]
    cc = lax.axis_index("c")
    ss = lax.axis_index("s")
    wid = ss * NC + cc
    t0 = T_TC + wid * T_PER_W
    base = (t0 - T_TC) * B

    pltpu.sync_copy(idx_hbm.at[pl.ds(base, T_PER_W * B)], idx_v)

    def issue_in(g, b):
        pltpu.async_copy(tbl_hbm.at[idx_v.at[pl.ds(g * K, K)]], pe[b], gsem[b])
        pltpu.async_copy(x_hbm.at[pl.ds(t0 + g * KT, KT)], xb[b], xsem[b])

    def process(g, b, prefetch_b):
        @pl.when(g >= 1)
        def _():
            pltpu.make_async_copy(xb[prefetch_b], out_hbm.at[pl.ds(t0, KT)],
                                  osem[prefetch_b]).wait()

        @pl.when(g + NBUF - 1 < NCHUNK)
        def _():
            issue_in(g + NBUF - 1, prefetch_b)

        pltpu.make_async_copy(x_hbm.at[pl.ds(t0, KT)], pe[b], gsem[b]).wait()
        pltpu.make_async_copy(x_hbm.at[pl.ds(t0, KT)], xb[b], xsem[b]).wait()

        def add_row(r, _):
            t = r // B
            bb = lax.rem(r, B)
            for j in range(D // L):
                sl = pl.ds(j * L, L)
                plsc.addupdate(xb[b].at[t, bb, sl], pe[b][r, sl])
            return 0

        lax.fori_loop(0, K, add_row, 0)
        pltpu.async_copy(xb[b], out_hbm.at[pl.ds(t0 + g * KT, KT)], osem[b])

    for b in range(NBUF - 1):
        issue_in(b, b)

    def group(p, _):
        for b in range(NBUF):
            g = p * NBUF + b
            process(g, b, (b + NBUF - 1) % NBUF)
        return 0

    lax.fori_loop(0, NCHUNK // NBUF, group, 0)
    for g in range(NCHUNK - NCHUNK % NBUF, NCHUNK):
        process(g, g % NBUF, (g + NBUF - 1) % NBUF)
    bl = (NCHUNK - 1) % NBUF
    pltpu.make_async_copy(xb[bl], out_hbm.at[pl.ds(t0, KT)], osem[bl]).wait()


def _sc_gather_add(idx, x, tbl):
    mesh = plsc.VectorSubcoreMesh(core_axis_name="c", subcore_axis_name="s")
    fn = functools.partial(
        pl.kernel,
        mesh=mesh,
        out_type=jax.ShapeDtypeStruct((T, B, D), jnp.float32),
        scratch_types=(
            [pltpu.VMEM((T_PER_W * B,), jnp.int32)]
            + [pltpu.VMEM((K, D), jnp.float32) for _ in range(NBUF)]
            + [pltpu.VMEM((KT, B, D), jnp.float32) for _ in range(NBUF)]
            + [pltpu.SemaphoreType.DMA for _ in range(3 * NBUF)]
        ),
    )(_sc_body)
    return fn(idx, x, tbl)


TSPAN = TBLK + 8


def _aligned_base(s):
    return pl.multiple_of((s >> 3) << 3, 8)


def _mm_body(starts_sm, c_ref, x_ref, tbl_ref, out_ref, slc, sems):
    i = pl.program_id(0)
    nblk = pl.num_programs(0)

    def issue(j, buf):
        for bb in range(B):
            c0 = _aligned_base(starts_sm[j, bb])
            pltpu.make_async_copy(tbl_ref.at[pl.ds(c0, TSPAN)],
                                  slc.at[buf, bb], sems.at[buf, bb]).start()

    @pl.when(i == 0)
    def _():
        issue(0, 0)

    @pl.when(i + 1 < nblk)
    def _():
        issue(i + 1, (i + 1) % 2)

    buf = i % 2
    for bb in range(B):
        pltpu.make_async_copy(tbl_ref.at[pl.ds(0, TSPAN)],
                              slc.at[buf, bb], sems.at[buf, bb]).wait()
    sidx = lax.broadcasted_iota(jnp.int32, (TBLK, TSPAN), 1)
    for bb in range(B):
        c0 = _aligned_base(starts_sm[i, bb])
        local = c_ref[:, bb] - c0
        onehot = (local[:, None] == sidx).astype(jnp.float32)
        pe = jnp.dot(onehot, slc[buf, bb],
                     preferred_element_type=jnp.float32,
                     precision=lax.Precision.HIGHEST)
        out_ref[:, bb, :] = x_ref[:, bb, :] + pe


def _mm_gather_add(starts, cum, x, tbl):
    nblk = T_TC // TBLK
    grid_spec = pltpu.PrefetchScalarGridSpec(
        num_scalar_prefetch=1,
        grid=(nblk,),
        in_specs=[
            pl.BlockSpec((TBLK, B), lambda i, s: (i, 0)),
            pl.BlockSpec((TBLK, B, D), lambda i, s: (i, 0, 0)),
            pl.BlockSpec(memory_space=pl.ANY),
        ],
        out_specs=pl.BlockSpec((TBLK, B, D), lambda i, s: (i, 0, 0)),
        scratch_shapes=[
            pltpu.VMEM((2, B, TSPAN, D), jnp.float32),
            pltpu.SemaphoreType.DMA((2, B)),
        ],
    )
    return pl.pallas_call(
        _mm_body,
        grid_spec=grid_spec,
        out_shape=jax.ShapeDtypeStruct((T_TC, B, D), jnp.float32),
    )(starts, cum, x, tbl)


def kernel(x_original, x_projected_to_d_model, pe_table):
    flags = x_original[:, :, -1]
    cum = _cumsum(flags)
    idx = cum[T_TC:].reshape(N_SC)
    sc_out = _sc_gather_add(idx, x_projected_to_d_model, pe_table)
    starts = cum[:T_TC:TBLK, :]
    tc_half = _mm_gather_add(starts, cum, x_projected_to_d_model, pe_table)
    return lax.dynamic_update_slice(sc_out, tc_half, (0, 0, 0))
